# baseline (speedup 1.0000x reference)
import functools

import jax
import jax.numpy as jnp
from jax import lax
from jax.experimental import pallas as pl
from jax.experimental.pallas import tpu as pltpu
from jax.experimental.pallas import tpu_sc as plsc

_L = 128
_NS = 16
_NC = 2


def _pad_up(v, m):
    return (v + m - 1) // m * m


def _divisor_below(n, cap):
    for b in range(cap, 0, -1):
        if n % b == 0:
            return b
    return 1


@functools.lru_cache(maxsize=None)
def _make_sc_pass(n_tab, R, Rt, n_acc, B):
    T = n_tab
    n_out = max(T, 1)
    slc = n_acc // _NS
    tpg = _NS // n_out
    Rtg = Rt * n_out
    slcg = n_acc // tpg
    n_chunks = Rtg // B

    mesh = plsc.VectorSubcoreMesh(core_axis_name="c", subcore_axis_name="s")

    scratch = [pltpu.VMEM((B, _L), jnp.int32)]
    if T:
        scratch.append(pltpu.VMEM((B, _L), jnp.int32))
        scratch.append(pltpu.VMEM((n_acc,), jnp.float32))
    scratch.append(pltpu.VMEM((n_acc,), jnp.float32))
    scratch.append(pltpu.SemaphoreType.DMA)

    out_type = jax.ShapeDtypeStruct((n_out * 2 * tpg * n_acc,), jnp.float32)

    def body(*refs):
        it = list(refs)
        dst2 = it.pop(0)
        src2 = it.pop(0) if T else None
        tabs = [it.pop(0) for _ in range(T)]
        z = it.pop(0)
        out = it.pop(0)
        idx_d = it.pop(0)
        idx_s = it.pop(0) if T else None
        tabv = it.pop(0) if T else None
        accv = it.pop(0)
        sem = it.pop(0)

        c = lax.axis_index("c")
        s = lax.axis_index("s")
        sg = lax.rem(s, tpg)
        g = s // tpg

        pltpu.async_copy(z, accv, sem).wait()
        for t in range(T):
            @pl.when(g == t)
            def _():
                pltpu.async_copy(tabs[t].at[pl.ds(c * n_acc, n_acc)], tabv,
                                 sem).wait()

        base = c * R + sg * Rtg
        one16 = jnp.full((16,), 1.0, jnp.float32)

        def chunk(i, carry):
            row = base + i * B
            pltpu.sync_copy(dst2.at[pl.ds(row, B)], idx_d)
            if T:
                pltpu.sync_copy(src2.at[pl.ds(row, B)], idx_s)
            for j in range(B):
                for k in range(_L // 16):
                    i16d = idx_d[j, pl.ds(k * 16, 16)]
                    if T:
                        i16s = idx_s[j, pl.ds(k * 16, 16)]
                        v16 = plsc.load_gather(tabv, [i16s])
                    else:
                        v16 = one16
                    plsc.addupdate_scatter(accv, [i16d], v16)
            return carry

        lax.fori_loop(0, n_chunks, chunk, 0)

        o = ((g * 2 + c) * tpg + sg) * n_acc
        pltpu.sync_copy(accv, out.at[pl.ds(o, n_acc)])

    return pl.kernel(body, out_type=out_type, scratch_types=scratch, mesh=mesh,
                     compiler_params=pltpu.CompilerParams(
                         needs_layout_passes=False))


def _reduce_parts(ref):
    acc = ref[:, 0, :]
    for t in range(1, ref.shape[1]):
        acc = acc + ref[:, t, :]
    return acc


def _tc_norm(cnt_p, x2):
    def body(cnt_ref, x_ref, dinv_ref, u_ref):
        dinv = lax.rsqrt(_reduce_parts(cnt_ref) + 1.0)
        dinv_ref[...] = dinv
        u_ref[...] = dinv * x_ref[...]

    sds = jax.ShapeDtypeStruct(x2.shape, jnp.float32)
    return pl.pallas_call(body, out_shape=(sds, sds))(cnt_p, x2)


def _tc_split(t1_p, dinv, x2):
    def body(t_ref, dinv_ref, x_ref, up_ref, uq_ref, d2p_ref, d2q_ref):
        dinv = dinv_ref[...]
        d2 = dinv * dinv
        sv = dinv * _reduce_parts(t_ref) + d2 * x_ref[...]
        p = jnp.maximum(sv, 0.0)
        q = sv - p
        up_ref[...] = dinv * p
        uq_ref[...] = dinv * q
        d2p_ref[...] = d2 * p
        d2q_ref[...] = d2 * q

    sds = jax.ShapeDtypeStruct(x2.shape, jnp.float32)
    return pl.pallas_call(body, out_shape=(sds,) * 4)(t1_p, dinv, x2)


def _tc_head(tp, tq, dinv, d2p, d2q, W1c, W2, b2c, fcc, phys2, fct, fcb2, N):
    n_acc = tp.shape[2]
    tpg = tp.shape[1]
    BN = _divisor_below(n_acc // _L, 32) * _L
    NB = n_acc // BN

    def body(tp_ref, tq_ref, dinv_ref, d2p_ref, d2q_ref, w1_ref, w2_ref,
             b2_ref, fcc_ref, ph_ref, fct_ref, fcb_ref, o_ref):
        j = pl.program_id(0)
        a = jnp.maximum(w1_ref[...], 0.0)
        cc = w1_ref[...] - a
        dn = (((0,), (0,)), ((), ()))
        A2 = lax.dot_general(w2_ref[...], a, dn,
                             preferred_element_type=jnp.float32)
        C2 = lax.dot_general(w2_ref[...], cc, dn,
                             preferred_element_type=jnp.float32)
        dinv = dinv_ref[...]
        P = dinv * _reduce_parts(tp_ref) + d2p_ref[...]
        Q = dinv * _reduce_parts(tq_ref) + d2q_ref[...]
        col = j * BN + lax.broadcasted_iota(jnp.int32, (1, BN), 1)
        msk = (col < N).astype(jnp.float32)

        total = jnp.zeros((1, 1), jnp.float32)
        for b in range(2):
            h = jnp.maximum(A2 * P[b:b + 1, :] + C2 * Q[b:b + 1, :]
                            + b2_ref[...], 0.0)
            h = h * msk
            hs = jnp.sum(h, axis=1, keepdims=True)
            total += jnp.sum(fcc_ref[:, b:b + 1] * hs).reshape(1, 1)

        @pl.when(j == 0)
        def _():
            o_ref[...] = ph_ref[...] * fct_ref[...] + fcb_ref[...]

        o_ref[...] += total

    full = lambda shape: pl.BlockSpec(shape, lambda j: (0,) * len(shape))
    blk = pl.BlockSpec((2, BN), lambda j: (0, j))
    blk3 = pl.BlockSpec((2, tpg, BN), lambda j: (0, 0, j))
    return pl.pallas_call(
        body,
        grid=(NB,),
        in_specs=[blk3, blk3, blk, blk, blk,
                  full((_L, 1)), full((_L, _L)), full((_L, 1)),
                  full((_L, 2)), full((1, 1)), full((1, 1)), full((1, 1))],
        out_specs=full((1, 1)),
        out_shape=jax.ShapeDtypeStruct((1, 1), jnp.float32),
    )(tp, tq, dinv, d2p, d2q, W1c, W2, b2c, fcc, phys2, fct, fcb2)


def kernel(mol_x, mol_edge_index, prot_x, prot_edge_index, phys_energy,
           W1, b1, W2, b2, fc_W, fc_b):
    N = mol_x.shape[0]
    E = mol_edge_index.shape[1]
    H = W1.shape[1]

    n_acc = _pad_up(N + 1, _L * _NS)
    e_pad = _pad_up(E, _L * _NS * 8)
    R = e_pad // _L
    Rt = R // _NS
    B = 8 * _divisor_below(Rt // 8, 7)

    def prep(ei, voff):
        srci = ei[0].astype(jnp.int32)
        dsti = ei[1].astype(jnp.int32)
        npad = e_pad - E
        srci = jnp.concatenate([srci + voff, jnp.full((npad,), N + voff, jnp.int32)])
        dsti = jnp.concatenate([dsti, jnp.full((npad,), N, jnp.int32)])
        return srci, dsti

    sm, dm = prep(mol_edge_index, 0)
    sp, dp = prep(prot_edge_index, 0)
    src2 = jnp.concatenate([sm, sp]).reshape(2 * R, _L)
    dst2 = jnp.concatenate([dm, dp]).reshape(2 * R, _L)

    zpad = jnp.zeros((n_acc - N,), jnp.float32)
    x2 = jnp.stack([jnp.concatenate([mol_x[:, 0], zpad]),
                    jnp.concatenate([prot_x[:, 0], zpad])])
    z = jnp.zeros((n_acc,), jnp.float32)

    cnt_p = _make_sc_pass(0, R, Rt, n_acc, B)(dst2, z).reshape(2, _NS, n_acc)
    dinv, u = _tc_norm(cnt_p, x2)
    t1_p = _make_sc_pass(1, R, Rt, n_acc, B)(dst2, src2, u.reshape(-1),
                                             z).reshape(2, _NS, n_acc)
    up, uq, d2p, d2q = _tc_split(t1_p, dinv, x2)
    tpq = _make_sc_pass(2, R, Rt, n_acc, B)(dst2, src2, up.reshape(-1),
                                            uq.reshape(-1), z)
    half = 2 * (_NS // 2) * n_acc
    tp = tpq[:half].reshape(2, _NS // 2, n_acc)
    tq = tpq[half:].reshape(2, _NS // 2, n_acc)

    W1c = W1.reshape(H, 1)
    b2c = b2.reshape(H, 1)
    fcc = jnp.stack([fc_W[0, :H], fc_W[0, H:2 * H]], axis=1)
    phys2 = phys_energy.reshape(1, 1)
    fct = fc_W[0, 2 * H:2 * H + 1].reshape(1, 1)
    fcb2 = fc_b.reshape(1, 1)

    return _tc_head(tp, tq, dinv, d2p, d2q, W1c, W2, b2c, fcc,
                    phys2, fct, fcb2, N)


# speedup vs baseline: 1.2693x; 1.2693x over previous
import functools

import jax
import jax.numpy as jnp
from jax import lax
from jax.experimental import pallas as pl
from jax.experimental.pallas import tpu as pltpu
from jax.experimental.pallas import tpu_sc as plsc

_L = 128
_NS = 16
_NC = 2


def _pad_up(v, m):
    return (v + m - 1) // m * m


def _divisor_below(n, cap):
    for b in range(cap, 0, -1):
        if n % b == 0:
            return b
    return 1


@functools.lru_cache(maxsize=None)
def _make_sc_pass(n_tab, R, Rt, n_acc, B):
    T = n_tab
    n_out = max(T, 1)
    slc = n_acc // _NS
    tpg = _NS // n_out
    Rtg = Rt * n_out
    slcg = n_acc // tpg
    n_chunks = Rtg // B

    mesh = plsc.VectorSubcoreMesh(core_axis_name="c", subcore_axis_name="s")

    scratch = [pltpu.VMEM((B, _L), jnp.int32)]
    if T:
        scratch.append(pltpu.VMEM((B, _L), jnp.int32))
    scratch.append(pltpu.VMEM((B, _L), jnp.float32))
    if T:
        scratch.append(pltpu.VMEM((n_acc,), jnp.float32))
    for _ in range(n_out):
        scratch.append(pltpu.VMEM_SHARED((n_acc,), jnp.float32))
    scratch.append(pltpu.SemaphoreType.DMA)
    scratch.append(pltpu.SemaphoreType.DMA)

    out_type = jax.ShapeDtypeStruct((n_out * 2 * n_acc,), jnp.float32)

    def body(*refs):
        it = list(refs)
        dst2 = it.pop(0)
        src2 = it.pop(0) if T else None
        tabs = [it.pop(0) for _ in range(T)]
        z = it.pop(0)
        ones = it.pop(0) if T == 0 else None
        out = it.pop(0)
        idx_d = it.pop(0)
        idx_s = it.pop(0) if T else None
        vals = it.pop(0)
        tabv = it.pop(0) if T else None
        accs = [it.pop(0) for _ in range(n_out)]
        sem = it.pop(0)
        sem2 = it.pop(0)

        c = lax.axis_index("c")
        s = lax.axis_index("s")
        off = s * slc
        sg = lax.rem(s, tpg)

        for a in accs:
            pltpu.sync_copy(z.at[pl.ds(off, slc)], a.at[pl.ds(off, slc)])
        if T == 0:
            pltpu.sync_copy(ones, vals)
        for t in range(T):
            @pl.when(s // tpg == t)
            def _():
                pltpu.async_copy(tabs[t].at[pl.ds(c * n_acc, n_acc)], tabv,
                                 sem).wait()
        plsc.subcore_barrier()

        base = c * R + sg * Rtg

        def make_chunk(acc):
            def chunk(i, carry):
                row = base + i * B
                pltpu.sync_copy(dst2.at[pl.ds(row, B)], idx_d)
                if T:
                    pltpu.sync_copy(src2.at[pl.ds(row, B)], idx_s)
                sds = []
                for j in range(B):
                    if T:
                        for k in range(_L // 16):
                            i16 = idx_s[j, pl.ds(k * 16, 16)]
                            vals[j, pl.ds(k * 16, 16)] = plsc.load_gather(
                                tabv, [i16])
                    sds.append(pltpu.async_copy(
                        vals.at[j], acc.at[idx_d.at[j]], sem2, add=True))
                for d in sds:
                    d.wait()
                return carry
            return chunk

        for t in range(n_out):
            @pl.when(s // tpg == t)
            def _():
                lax.fori_loop(0, n_chunks, make_chunk(accs[t]), 0)
        plsc.subcore_barrier()

        for t in range(n_out):
            o = (t * 2 + c) * n_acc + off
            pltpu.sync_copy(accs[t].at[pl.ds(off, slc)], out.at[pl.ds(o, slc)])

    return pl.kernel(body, out_type=out_type, scratch_types=scratch, mesh=mesh,
                     compiler_params=pltpu.CompilerParams(
                         needs_layout_passes=False))


def _tc_norm(cnt, x2):
    def body(cnt_ref, x_ref, dinv_ref, u_ref):
        dinv = lax.rsqrt(cnt_ref[...] + 1.0)
        dinv_ref[...] = dinv
        u_ref[...] = dinv * x_ref[...]

    sds = jax.ShapeDtypeStruct(x2.shape, jnp.float32)
    return pl.pallas_call(body, out_shape=(sds, sds))(cnt, x2)


def _tc_split(t1, dinv, x2):
    def body(t_ref, dinv_ref, x_ref, up_ref, uq_ref, d2p_ref, d2q_ref):
        dinv = dinv_ref[...]
        d2 = dinv * dinv
        sv = dinv * t_ref[...] + d2 * x_ref[...]
        p = jnp.maximum(sv, 0.0)
        q = sv - p
        up_ref[...] = dinv * p
        uq_ref[...] = dinv * q
        d2p_ref[...] = d2 * p
        d2q_ref[...] = d2 * q

    sds = jax.ShapeDtypeStruct(x2.shape, jnp.float32)
    return pl.pallas_call(body, out_shape=(sds,) * 4)(t1, dinv, x2)


def _tc_head(tp, tq, dinv, d2p, d2q, W1c, W2, b2c, fcc, phys2, fct, fcb2, N):
    n_acc = tp.shape[1]
    BN = _divisor_below(n_acc // _L, 32) * _L
    NB = n_acc // BN

    def body(tp_ref, tq_ref, dinv_ref, d2p_ref, d2q_ref, w1_ref, w2_ref,
             b2_ref, fcc_ref, ph_ref, fct_ref, fcb_ref, o_ref):
        j = pl.program_id(0)
        a = jnp.maximum(w1_ref[...], 0.0)
        cc = w1_ref[...] - a
        dn = (((0,), (0,)), ((), ()))
        A2 = lax.dot_general(w2_ref[...], a, dn,
                             preferred_element_type=jnp.float32)
        C2 = lax.dot_general(w2_ref[...], cc, dn,
                             preferred_element_type=jnp.float32)
        dinv = dinv_ref[...]
        P = dinv * tp_ref[...] + d2p_ref[...]
        Q = dinv * tq_ref[...] + d2q_ref[...]
        col = j * BN + lax.broadcasted_iota(jnp.int32, (1, BN), 1)
        msk = (col < N).astype(jnp.float32)

        total = jnp.zeros((1, 1), jnp.float32)
        for b in range(2):
            h = jnp.maximum(A2 * P[b:b + 1, :] + C2 * Q[b:b + 1, :]
                            + b2_ref[...], 0.0)
            h = h * msk
            hs = jnp.sum(h, axis=1, keepdims=True)
            total += jnp.sum(fcc_ref[:, b:b + 1] * hs).reshape(1, 1)

        @pl.when(j == 0)
        def _():
            o_ref[...] = ph_ref[...] * fct_ref[...] + fcb_ref[...]

        o_ref[...] += total

    full = lambda shape: pl.BlockSpec(shape, lambda j: (0,) * len(shape))
    blk = pl.BlockSpec((2, BN), lambda j: (0, j))
    return pl.pallas_call(
        body,
        grid=(NB,),
        in_specs=[blk, blk, blk, blk, blk,
                  full((_L, 1)), full((_L, _L)), full((_L, 1)),
                  full((_L, 2)), full((1, 1)), full((1, 1)), full((1, 1))],
        out_specs=full((1, 1)),
        out_shape=jax.ShapeDtypeStruct((1, 1), jnp.float32),
    )(tp, tq, dinv, d2p, d2q, W1c, W2, b2c, fcc, phys2, fct, fcb2)


def kernel(mol_x, mol_edge_index, prot_x, prot_edge_index, phys_energy,
           W1, b1, W2, b2, fc_W, fc_b):
    N = mol_x.shape[0]
    E = mol_edge_index.shape[1]
    H = W1.shape[1]

    n_acc = _pad_up(N + 1, _L * _NS)
    e_pad = _pad_up(E, _L * _NS * 8)
    R = e_pad // _L
    Rt = R // _NS
    B = 8 * _divisor_below(Rt // 8, 7)

    def prep(ei, voff):
        srci = ei[0].astype(jnp.int32)
        dsti = ei[1].astype(jnp.int32)
        npad = e_pad - E
        srci = jnp.concatenate([srci + voff, jnp.full((npad,), N + voff, jnp.int32)])
        dsti = jnp.concatenate([dsti, jnp.full((npad,), N, jnp.int32)])
        return srci, dsti

    sm, dm = prep(mol_edge_index, 0)
    sp, dp = prep(prot_edge_index, 0)
    src2 = jnp.concatenate([sm, sp]).reshape(2 * R, _L)
    dst2 = jnp.concatenate([dm, dp]).reshape(2 * R, _L)

    zpad = jnp.zeros((n_acc - N,), jnp.float32)
    x2 = jnp.stack([jnp.concatenate([mol_x[:, 0], zpad]),
                    jnp.concatenate([prot_x[:, 0], zpad])])
    z = jnp.zeros((n_acc,), jnp.float32)
    ones = jnp.ones((B, _L), jnp.float32)

    cnt = _make_sc_pass(0, R, Rt, n_acc, B)(dst2, z, ones).reshape(2, n_acc)
    dinv, u = _tc_norm(cnt, x2)
    t1 = _make_sc_pass(1, R, Rt, n_acc, B)(dst2, src2, u.reshape(-1), z)
    up, uq, d2p, d2q = _tc_split(t1.reshape(2, n_acc), dinv, x2)
    tpq = _make_sc_pass(2, R, Rt, n_acc, B)(dst2, src2, up.reshape(-1),
                                            uq.reshape(-1), z)
    tp = tpq[:2 * n_acc].reshape(2, n_acc)
    tq = tpq[2 * n_acc:].reshape(2, n_acc)

    W1c = W1.reshape(H, 1)
    b2c = b2.reshape(H, 1)
    fcc = jnp.stack([fc_W[0, :H], fc_W[0, H:2 * H]], axis=1)
    phys2 = phys_energy.reshape(1, 1)
    fct = fc_W[0, 2 * H:2 * H + 1].reshape(1, 1)
    fcb2 = fc_b.reshape(1, 1)

    return _tc_head(tp, tq, dinv, d2p, d2q, W1c, W2, b2c, fcc,
                    phys2, fct, fcb2, N)


# speedup vs baseline: 1.3413x; 1.0567x over previous
import functools

import jax
import jax.numpy as jnp
from jax import lax
from jax.experimental import pallas as pl
from jax.experimental.pallas import tpu as pltpu
from jax.experimental.pallas import tpu_sc as plsc

_L = 128
_NS = 16
_NC = 2
_V = 16


def _pad_up(v, m):
    return (v + m - 1) // m * m


def _divisor_below(n, cap):
    for b in range(cap, 0, -1):
        if n % b == 0:
            return b
    return 1


@functools.lru_cache(maxsize=None)
def _make_sc_fused(R, Rt, n_acc, B):
    slc = n_acc // _NS
    n_chunks = Rt // B
    tpg = _NS // 2
    Rtg = Rt * 2
    n_chunks_g = Rtg // B
    n_vec = slc // _V

    mesh = plsc.VectorSubcoreMesh(core_axis_name="c", subcore_axis_name="s")

    scratch = [
        pltpu.VMEM((B, _L), jnp.int32),
        pltpu.VMEM((B, _L), jnp.int32),
        pltpu.VMEM((B, _L), jnp.float32),
        pltpu.VMEM((n_acc,), jnp.float32),
        pltpu.VMEM((slc,), jnp.float32),
        pltpu.VMEM((slc,), jnp.float32),
        pltpu.VMEM((slc,), jnp.float32),
        pltpu.VMEM((slc,), jnp.float32),
        pltpu.VMEM((slc,), jnp.float32),
        pltpu.VMEM_SHARED((n_acc,), jnp.float32),
        pltpu.VMEM_SHARED((n_acc,), jnp.float32),
        pltpu.VMEM_SHARED((n_acc,), jnp.float32),
        pltpu.VMEM_SHARED((n_acc,), jnp.float32),
        pltpu.VMEM_SHARED((n_acc,), jnp.float32),
        pltpu.VMEM_SHARED((n_acc,), jnp.float32),
        pltpu.SemaphoreType.DMA,
        pltpu.SemaphoreType.DMA,
    ]

    out_type = jax.ShapeDtypeStruct((5 * 2 * n_acc,), jnp.float32)

    def body(dst2, src2, x2, z, ones, out, idx_d, idx_s, vals, tabv, xbuf,
             cbuf, dbuf, pbuf, qbuf, acc_cnt, acc_t1, acc_tp, acc_tq,
             up_s, uq_s, sem, sem2):
        c = lax.axis_index("c")
        s = lax.axis_index("s")
        off = s * slc
        g = s // tpg
        sg = lax.rem(s, tpg)

        for a in (acc_cnt, acc_t1, acc_tp, acc_tq):
            pltpu.sync_copy(z.at[pl.ds(off, slc)], a.at[pl.ds(off, slc)])
        pltpu.sync_copy(x2.at[pl.ds(c * n_acc + off, slc)], xbuf)
        pltpu.sync_copy(ones, vals)

        def scatter_chunks(base, count, acc, gather):
            def chunk(i, carry):
                row = base + i * B
                pltpu.sync_copy(dst2.at[pl.ds(row, B)], idx_d)
                if gather:
                    pltpu.sync_copy(src2.at[pl.ds(row, B)], idx_s)
                sds = []
                for j in range(B):
                    if gather:
                        for k in range(_L // _V):
                            i16 = idx_s[j, pl.ds(k * _V, _V)]
                            vals[j, pl.ds(k * _V, _V)] = plsc.load_gather(
                                tabv, [i16])
                    sds.append(pltpu.async_copy(
                        vals.at[j], acc.at[idx_d.at[j]], sem2, add=True))
                for d in sds:
                    d.wait()
                return carry
            lax.fori_loop(0, count, chunk, 0)

        scatter_chunks(c * R + s * Rt, n_chunks, acc_cnt, False)
        plsc.subcore_barrier()

        pltpu.sync_copy(acc_cnt.at[pl.ds(off, slc)], cbuf)
        def norm_vec(i, carry):
            o16 = i * _V
            xv = cbuf[pl.ds(o16, _V)] + 1.0
            yi = lax.bitcast_convert_type(xv, jnp.int32)
            yi = 0x5F3759DF - lax.shift_right_logical(yi, 1)
            dv = lax.bitcast_convert_type(yi, jnp.float32)
            for _ in range(3):
                dv = dv * (1.5 - 0.5 * xv * dv * dv)
            dbuf[pl.ds(o16, _V)] = dv
            pbuf[pl.ds(o16, _V)] = dv * xbuf[pl.ds(o16, _V)]
            return carry
        lax.fori_loop(0, n_vec, norm_vec, 0)
        pltpu.sync_copy(pbuf, up_s.at[pl.ds(off, slc)])
        plsc.subcore_barrier()

        pltpu.sync_copy(up_s, tabv)
        plsc.subcore_barrier()
        scatter_chunks(c * R + s * Rt, n_chunks, acc_t1, True)
        plsc.subcore_barrier()

        pltpu.sync_copy(acc_t1.at[pl.ds(off, slc)], cbuf)
        def split_vec(i, carry):
            o16 = i * _V
            dv = dbuf[pl.ds(o16, _V)]
            d2 = dv * dv
            sv = dv * cbuf[pl.ds(o16, _V)] + d2 * xbuf[pl.ds(o16, _V)]
            p = jnp.maximum(sv, 0.0)
            q = sv - p
            pbuf[pl.ds(o16, _V)] = dv * p
            qbuf[pl.ds(o16, _V)] = dv * q
            return carry
        lax.fori_loop(0, n_vec, split_vec, 0)
        pltpu.sync_copy(pbuf, up_s.at[pl.ds(off, slc)])
        pltpu.sync_copy(qbuf, uq_s.at[pl.ds(off, slc)])
        def selfloop_vec(i, carry):
            o16 = i * _V
            dv = dbuf[pl.ds(o16, _V)]
            pbuf[pl.ds(o16, _V)] = dv * pbuf[pl.ds(o16, _V)]
            qbuf[pl.ds(o16, _V)] = dv * qbuf[pl.ds(o16, _V)]
            return carry
        lax.fori_loop(0, n_vec, selfloop_vec, 0)
        pltpu.sync_copy(dbuf, out.at[pl.ds((2 * 2 + c) * n_acc + off, slc)])
        pltpu.sync_copy(pbuf, out.at[pl.ds((3 * 2 + c) * n_acc + off, slc)])
        pltpu.sync_copy(qbuf, out.at[pl.ds((4 * 2 + c) * n_acc + off, slc)])
        plsc.subcore_barrier()

        @pl.when(g == 0)
        def _():
            pltpu.sync_copy(up_s, tabv)
        @pl.when(g == 1)
        def _():
            pltpu.sync_copy(uq_s, tabv)
        plsc.subcore_barrier()
        @pl.when(g == 0)
        def _():
            scatter_chunks(c * R + sg * Rtg, n_chunks_g, acc_tp, True)
        @pl.when(g == 1)
        def _():
            scatter_chunks(c * R + sg * Rtg, n_chunks_g, acc_tq, True)
        plsc.subcore_barrier()

        for t, buf in enumerate((acc_tp, acc_tq)):
            o = (t * 2 + c) * n_acc + off
            pltpu.sync_copy(buf.at[pl.ds(off, slc)], out.at[pl.ds(o, slc)])

    return pl.kernel(body, out_type=out_type, scratch_types=scratch,
                     mesh=mesh,
                     compiler_params=pltpu.CompilerParams(
                         needs_layout_passes=False))


def _tc_head(tp, tq, dinv, d2p, d2q, W1c, W2, b2c, fcc, phys2, fct, fcb2, N):
    n_acc = tp.shape[1]
    BN = _divisor_below(n_acc // _L, 32) * _L
    NB = n_acc // BN

    def body(tp_ref, tq_ref, dinv_ref, d2p_ref, d2q_ref, w1_ref, w2_ref,
             b2_ref, fcc_ref, ph_ref, fct_ref, fcb_ref, o_ref):
        j = pl.program_id(0)
        a = jnp.maximum(w1_ref[...], 0.0)
        cc = w1_ref[...] - a
        dn = (((0,), (0,)), ((), ()))
        A2 = lax.dot_general(w2_ref[...], a, dn,
                             preferred_element_type=jnp.float32)
        C2 = lax.dot_general(w2_ref[...], cc, dn,
                             preferred_element_type=jnp.float32)
        dinv = dinv_ref[...]
        P = dinv * tp_ref[...] + d2p_ref[...]
        Q = dinv * tq_ref[...] + d2q_ref[...]
        col = j * BN + lax.broadcasted_iota(jnp.int32, (1, BN), 1)
        msk = (col < N).astype(jnp.float32)

        total = jnp.zeros((1, 1), jnp.float32)
        for b in range(2):
            h = jnp.maximum(A2 * P[b:b + 1, :] + C2 * Q[b:b + 1, :]
                            + b2_ref[...], 0.0)
            h = h * msk
            hs = jnp.sum(h, axis=1, keepdims=True)
            total += jnp.sum(fcc_ref[:, b:b + 1] * hs).reshape(1, 1)

        @pl.when(j == 0)
        def _():
            o_ref[...] = ph_ref[...] * fct_ref[...] + fcb_ref[...]

        o_ref[...] += total

    full = lambda shape: pl.BlockSpec(shape, lambda j: (0,) * len(shape))
    blk = pl.BlockSpec((2, BN), lambda j: (0, j))
    return pl.pallas_call(
        body,
        grid=(NB,),
        in_specs=[blk, blk, blk, blk, blk,
                  full((_L, 1)), full((_L, _L)), full((_L, 1)),
                  full((_L, 2)), full((1, 1)), full((1, 1)), full((1, 1))],
        out_specs=full((1, 1)),
        out_shape=jax.ShapeDtypeStruct((1, 1), jnp.float32),
    )(tp, tq, dinv, d2p, d2q, W1c, W2, b2c, fcc, phys2, fct, fcb2)


def kernel(mol_x, mol_edge_index, prot_x, prot_edge_index, phys_energy,
           W1, b1, W2, b2, fc_W, fc_b):
    N = mol_x.shape[0]
    E = mol_edge_index.shape[1]
    H = W1.shape[1]

    n_acc = _pad_up(N + 1, _L * _NS)
    e_pad = _pad_up(E, _L * _NS * 8)
    R = e_pad // _L
    Rt = R // _NS
    B = 8 * _divisor_below(Rt // 8, 7)

    def prep(ei):
        srci = ei[0].astype(jnp.int32)
        dsti = ei[1].astype(jnp.int32)
        npad = e_pad - E
        srci = jnp.concatenate([srci, jnp.full((npad,), N, jnp.int32)])
        dsti = jnp.concatenate([dsti, jnp.full((npad,), N, jnp.int32)])
        return srci, dsti

    sm, dm = prep(mol_edge_index)
    sp, dp = prep(prot_edge_index)
    src2 = jnp.concatenate([sm, sp]).reshape(2 * R, _L)
    dst2 = jnp.concatenate([dm, dp]).reshape(2 * R, _L)

    zpad = jnp.zeros((n_acc - N,), jnp.float32)
    x2 = jnp.concatenate([mol_x[:, 0], zpad, prot_x[:, 0], zpad])
    z = jnp.zeros((n_acc,), jnp.float32)
    ones = jnp.ones((B, _L), jnp.float32)

    res = _make_sc_fused(R, Rt, n_acc, B)(dst2, src2, x2, z, ones)
    res = res.reshape(5, 2, n_acc)
    tp, tq, dinv, d2p, d2q = (res[i] for i in range(5))

    W1c = W1.reshape(H, 1)
    b2c = b2.reshape(H, 1)
    fcc = jnp.stack([fc_W[0, :H], fc_W[0, H:2 * H]], axis=1)
    phys2 = phys_energy.reshape(1, 1)
    fct = fc_W[0, 2 * H:2 * H + 1].reshape(1, 1)
    fcb2 = fc_b.reshape(1, 1)

    return _tc_head(tp, tq, dinv, d2p, d2q, W1c, W2, b2c, fcc,
                    phys2, fct, fcb2, N)
